# baseline (speedup 1.0000x reference)
import functools

import jax
import jax.numpy as jnp
from jax import lax
from jax.experimental import pallas as pl
from jax.experimental.pallas import tpu as pltpu
from jax.experimental.pallas import tpu_sc as plsc

N_NODES = 10000
D_FEAT = 128
N_EDGES = 320000

NC = 2
NS = 16
NW = NC * NS
EDGES_PER_W = N_EDGES // NW
CHUNK = 80
NCHUNK = EDGES_PER_W // CHUNK
NPAIR = (NCHUNK - 1) // 2
ZROWS = 25
ROWS_PER_TILE = N_NODES // NS
WROWS = 125
LANES = 16

_mesh = plsc.VectorSubcoreMesh(core_axis_name="c", subcore_axis_name="s")


@functools.partial(
    pl.kernel,
    out_type=jax.ShapeDtypeStruct((NC, N_NODES, D_FEAT), jnp.float32),
    mesh=_mesh,
    compiler_params=pltpu.CompilerParams(use_tc_tiling_on_sc=False),
    scratch_types=[
        pltpu.VMEM((NCHUNK, CHUNK), jnp.int32),
        pltpu.VMEM((NCHUNK, CHUNK), jnp.int32),
        pltpu.VMEM((CHUNK, D_FEAT), jnp.float32),
        pltpu.VMEM((CHUNK, D_FEAT), jnp.float32),
        pltpu.VMEM_SHARED((N_NODES, D_FEAT), jnp.float32),
        pltpu.SemaphoreType.DMA,
        pltpu.SemaphoreType.DMA,
        pltpu.SemaphoreType.DMA,
        pltpu.SemaphoreType.DMA,
    ],
)
def _gather_scatter_sc(x_hbm, ei_hbm, out_hbm,
                       row_v, col_v, xa, xb, acc, sga, sgb, ssa, ssb):
    c = lax.axis_index("c")
    s = lax.axis_index("s")
    wid = s * NC + c

    with jax.named_scope("stage_idx"):
        idx_a = pltpu.async_copy(ei_hbm.at[0, wid], row_v, sga)
        idx_b = pltpu.async_copy(ei_hbm.at[1, wid], col_v, sgb)

    zv = jnp.zeros((LANES,), jnp.float32)

    with jax.named_scope("zero_acc"):
        def _zrow(i, _):
            def _zcol(k, _):
                xa[i, pl.ds(k * LANES, LANES)] = zv
                return 0
            return lax.fori_loop(0, D_FEAT // LANES, _zcol, 0)

        lax.fori_loop(0, ZROWS, _zrow, 0)

        base_row = s * ROWS_PER_TILE
        zsrc = xa.at[pl.ds(0, ZROWS)]

        def _zacc(j, _):
            pltpu.async_copy(zsrc, acc.at[pl.ds(base_row + j * ZROWS, ZROWS)],
                             ssa)
            return 0

        nz = ROWS_PER_TILE // ZROWS
        lax.fori_loop(0, nz, _zacc, 0)

        def _zdrain(j, _):
            pltpu.make_async_copy(
                zsrc, acc.at[pl.ds(base_row, ZROWS)], ssa).wait()
            return 0

        lax.fori_loop(0, nz, _zdrain, 0)
        idx_a.wait()
        idx_b.wait()
        plsc.subcore_barrier()

    def _g(chunk, buf, sem):
        return pltpu.async_copy(x_hbm.at[row_v.at[chunk]], buf, sem)

    def _s(chunk, buf, sem):
        return pltpu.async_copy(buf, acc.at[col_v.at[chunk]], sem,
                                add=True)

    def _gwait(chunk, buf, sem):
        pltpu.make_async_copy(x_hbm.at[row_v.at[chunk]], buf, sem).wait()

    def _swait(chunk, buf, sem):
        pltpu.make_async_copy(buf, acc.at[col_v.at[chunk]], sem).wait()

    with jax.named_scope("edge_loop"):
        _g(0, xa, sga)
        _g(1, xb, sgb)

        def _pair(j, _):
            ca = 2 * j
            cb = 2 * j + 1
            _gwait(ca, xa, sga)
            _s(ca, xa, ssa)
            _gwait(cb, xb, sgb)
            _s(cb, xb, ssb)
            _swait(ca, xa, ssa)
            _g(ca + 2, xa, sga)

            @pl.when(j < NPAIR - 1)
            def _():
                _swait(cb, xb, ssb)
                _g(cb + 2, xb, sgb)

            return 0

        lax.fori_loop(0, NPAIR, _pair, 0)
        _swait(NCHUNK - 2, xb, ssb)
        _gwait(NCHUNK - 1, xa, sga)
        _s(NCHUNK - 1, xa, ssa)
        _swait(NCHUNK - 1, xa, ssa)
        plsc.subcore_barrier()

    with jax.named_scope("writeout"):
        def _wout(j, _):
            r0 = base_row + j * WROWS
            pltpu.async_copy(acc.at[pl.ds(r0, WROWS)],
                             out_hbm.at[c, pl.ds(r0, WROWS)], ssa)
            return 0

        nw = ROWS_PER_TILE // WROWS
        lax.fori_loop(0, nw, _wout, 0)

        def _wdrain(j, _):
            pltpu.make_async_copy(
                acc.at[pl.ds(base_row, WROWS)],
                out_hbm.at[c, pl.ds(base_row, WROWS)], ssa).wait()
            return 0

        lax.fori_loop(0, nw, _wdrain, 0)


def _combine_body(p_ref, o_ref):
    o_ref[...] = p_ref[0] + p_ref[1]


_combine_tc = pl.pallas_call(
    _combine_body,
    grid=(10,),
    in_specs=[pl.BlockSpec((2, N_NODES // 10, D_FEAT), lambda i: (0, i, 0))],
    out_specs=pl.BlockSpec((N_NODES // 10, D_FEAT), lambda i: (i, 0)),
    out_shape=jax.ShapeDtypeStruct((N_NODES, D_FEAT), jnp.float32),
)


def kernel(x, edge_index):
    ei = edge_index.astype(jnp.int32).reshape(2, NW, NCHUNK, CHUNK)
    partials = _gather_scatter_sc(x, ei)
    return _combine_tc(partials)


# speedup vs baseline: 1.2565x; 1.2565x over previous
import functools

import jax
import jax.numpy as jnp
from jax import lax
from jax.experimental import pallas as pl
from jax.experimental.pallas import tpu as pltpu
from jax.experimental.pallas import tpu_sc as plsc

N_NODES = 10000
D_FEAT = 128
N_EDGES = 320000

NC = 2
NS = 16
NW = NC * NS
EDGES_PER_W = N_EDGES // NW
CHUNK = 80
NCHUNK = EDGES_PER_W // CHUNK
NPAIR = (NCHUNK - 1) // 2
ZROWS = 25
ROWS_PER_TILE = N_NODES // NS
WROWS = 125
LANES = 16

_mesh = plsc.VectorSubcoreMesh(core_axis_name="c", subcore_axis_name="s")


@functools.partial(
    pl.kernel,
    out_type=jax.ShapeDtypeStruct((NC, N_NODES, D_FEAT), jnp.float32),
    mesh=_mesh,
    compiler_params=pltpu.CompilerParams(use_tc_tiling_on_sc=False),
    scratch_types=[
        pltpu.VMEM((NCHUNK, CHUNK), jnp.int32),
        pltpu.VMEM((NCHUNK, CHUNK), jnp.int32),
        pltpu.VMEM((CHUNK, D_FEAT), jnp.float32),
        pltpu.VMEM((CHUNK, D_FEAT), jnp.float32),
        pltpu.VMEM_SHARED((N_NODES, D_FEAT), jnp.float32),
        pltpu.SemaphoreType.DMA,
        pltpu.SemaphoreType.DMA,
        pltpu.SemaphoreType.DMA,
        pltpu.SemaphoreType.DMA,
    ],
)
def _gather_scatter_sc(x_hbm, ei_hbm, out_hbm,
                       row_v, col_v, xa, xb, acc, sga, sgb, ssa, ssb):
    c = lax.axis_index("c")
    s = lax.axis_index("s")
    wid = s * NC + c

    with jax.named_scope("stage_idx"):
        idx_a = pltpu.async_copy(ei_hbm.at[0, wid], row_v, sga)
        idx_b = pltpu.async_copy(ei_hbm.at[1, wid], col_v, sgb)

    zv = jnp.zeros((LANES,), jnp.float32)

    with jax.named_scope("zero_acc"):
        def _zrow(i, _):
            def _zcol(k, _):
                xa[i, pl.ds(k * LANES, LANES)] = zv
                return 0
            return lax.fori_loop(0, D_FEAT // LANES, _zcol, 0)

        lax.fori_loop(0, ZROWS, _zrow, 0)

        base_row = s * ROWS_PER_TILE
        zsrc = xa.at[pl.ds(0, ZROWS)]

        def _zacc(j, _):
            pltpu.async_copy(zsrc, acc.at[pl.ds(base_row + j * ZROWS, ZROWS)],
                             ssa)
            return 0

        nz = ROWS_PER_TILE // ZROWS
        lax.fori_loop(0, nz, _zacc, 0)

        def _zdrain(j, _):
            pltpu.make_async_copy(
                zsrc, acc.at[pl.ds(base_row, ZROWS)], ssa).wait()
            return 0

        lax.fori_loop(0, nz, _zdrain, 0)
        idx_a.wait()
        idx_b.wait()
        plsc.subcore_barrier()

    def _g(chunk, buf, sem):
        return pltpu.async_copy(x_hbm.at[row_v.at[chunk]], buf, sem)

    def _gwait(chunk, buf, sem):
        pltpu.make_async_copy(x_hbm.at[row_v.at[chunk]], buf, sem).wait()

    with jax.named_scope("edge_loop"):
        _g(0, xa, sga)

        def _pair(j, _):
            ca = 2 * j
            cb = 2 * j + 1
            _g(cb, xb, sgb)
            _gwait(ca, xa, sga)
            pltpu.sync_copy(xa, acc.at[col_v.at[ca]], add=True)
            _g(ca + 2, xa, sga)
            _gwait(cb, xb, sgb)
            pltpu.sync_copy(xb, acc.at[col_v.at[cb]], add=True)
            return 0

        lax.fori_loop(0, NPAIR, _pair, 0)
        _gwait(NCHUNK - 1, xa, sga)
        pltpu.sync_copy(xa, acc.at[col_v.at[NCHUNK - 1]], add=True)
        plsc.subcore_barrier()

    with jax.named_scope("writeout"):
        def _wout(j, _):
            r0 = base_row + j * WROWS
            pltpu.async_copy(acc.at[pl.ds(r0, WROWS)],
                             out_hbm.at[c, pl.ds(r0, WROWS)], ssa)
            return 0

        nw = ROWS_PER_TILE // WROWS
        lax.fori_loop(0, nw, _wout, 0)

        def _wdrain(j, _):
            pltpu.make_async_copy(
                acc.at[pl.ds(base_row, WROWS)],
                out_hbm.at[c, pl.ds(base_row, WROWS)], ssa).wait()
            return 0

        lax.fori_loop(0, nw, _wdrain, 0)


def _combine_body(p_ref, o_ref):
    o_ref[...] = p_ref[0] + p_ref[1]


_combine_tc = pl.pallas_call(
    _combine_body,
    grid=(10,),
    in_specs=[pl.BlockSpec((2, N_NODES // 10, D_FEAT), lambda i: (0, i, 0))],
    out_specs=pl.BlockSpec((N_NODES // 10, D_FEAT), lambda i: (i, 0)),
    out_shape=jax.ShapeDtypeStruct((N_NODES, D_FEAT), jnp.float32),
)


def kernel(x, edge_index):
    ei = edge_index.astype(jnp.int32).reshape(2, NW, NCHUNK, CHUNK)
    partials = _gather_scatter_sc(x, ei)
    return _combine_tc(partials)


# speedup vs baseline: 1.4447x; 1.1498x over previous
import functools

import jax
import jax.numpy as jnp
from jax import lax
from jax.experimental import pallas as pl
from jax.experimental.pallas import tpu as pltpu
from jax.experimental.pallas import tpu_sc as plsc

N_NODES = 10000
D_FEAT = 128
N_EDGES = 320000

NC = 2
NS = 16
NW = NC * NS
EDGES_PER_W = N_EDGES // NW
CHUNK = 80
NCHUNK = EDGES_PER_W // CHUNK
NPAIR = (NCHUNK - 1) // 2
ZROWS = 25
ROWS_PER_TILE = N_NODES // NS
WROWS = 125
LANES = 16

_mesh = plsc.VectorSubcoreMesh(core_axis_name="c", subcore_axis_name="s")


@functools.partial(
    pl.kernel,
    out_type=jax.ShapeDtypeStruct((NC, N_NODES, D_FEAT), jnp.float32),
    mesh=_mesh,
    compiler_params=pltpu.CompilerParams(use_tc_tiling_on_sc=False),
    scratch_types=[
        pltpu.VMEM((NCHUNK, CHUNK), jnp.int32),
        pltpu.VMEM((NCHUNK, CHUNK), jnp.int32),
        pltpu.VMEM((CHUNK, D_FEAT), jnp.float32),
        pltpu.VMEM((CHUNK, D_FEAT), jnp.float32),
        pltpu.VMEM((CHUNK, D_FEAT), jnp.float32),
        pltpu.VMEM_SHARED((N_NODES, D_FEAT), jnp.float32),
        pltpu.SemaphoreType.DMA,
        pltpu.SemaphoreType.DMA,
        pltpu.SemaphoreType.DMA,
        pltpu.SemaphoreType.DMA,
        pltpu.SemaphoreType.DMA,
        pltpu.SemaphoreType.DMA,
    ],
)
def _gather_scatter_sc(x_hbm, ei_hbm, out_hbm,
                       row_v, col_v, xa, xb, xc, acc,
                       sga, sgb, sgc, ssa, ssb, ssc):
    c = lax.axis_index("c")
    s = lax.axis_index("s")
    wid = s * NC + c

    with jax.named_scope("stage_idx"):
        idx_a = pltpu.async_copy(ei_hbm.at[0, wid], row_v, sga)
        idx_b = pltpu.async_copy(ei_hbm.at[1, wid], col_v, sgb)

    zv = jnp.zeros((LANES,), jnp.float32)

    with jax.named_scope("zero_acc"):
        def _zrow(i, _):
            def _zcol(k, _):
                xa[i, pl.ds(k * LANES, LANES)] = zv
                return 0
            return lax.fori_loop(0, D_FEAT // LANES, _zcol, 0)

        lax.fori_loop(0, ZROWS, _zrow, 0)

        base_row = s * ROWS_PER_TILE
        zsrc = xa.at[pl.ds(0, ZROWS)]

        def _zacc(j, _):
            pltpu.async_copy(zsrc, acc.at[pl.ds(base_row + j * ZROWS, ZROWS)],
                             ssa)
            return 0

        nz = ROWS_PER_TILE // ZROWS
        lax.fori_loop(0, nz, _zacc, 0)

        def _zdrain(j, _):
            pltpu.make_async_copy(
                zsrc, acc.at[pl.ds(base_row, ZROWS)], ssa).wait()
            return 0

        lax.fori_loop(0, nz, _zdrain, 0)
        idx_a.wait()
        idx_b.wait()
        plsc.subcore_barrier()

    def _g(chunk, buf, sem):
        return pltpu.async_copy(x_hbm.at[row_v.at[chunk]], buf, sem)

    def _gwait(chunk, buf, sem):
        pltpu.make_async_copy(x_hbm.at[row_v.at[chunk]], buf, sem).wait()

    def _s(chunk, buf, sem):
        return pltpu.async_copy(buf, acc.at[col_v.at[chunk]], sem, add=True)

    def _swait(chunk, buf, sem):
        pltpu.make_async_copy(buf, acc.at[col_v.at[chunk]], sem).wait()

    bufs = (xa, xb, xc)
    gsems = (sga, sgb, sgc)
    ssems = (ssa, ssb, ssc)

    with jax.named_scope("edge_loop"):
        _g(0, xa, sga)
        _g(1, xb, sgb)
        _gwait(0, xa, sga)
        _s(0, xa, ssa)
        _g(2, xc, sgc)
        _gwait(1, xb, sgb)
        _s(1, xb, ssb)
        _swait(0, xa, ssa)
        _g(3, xa, sga)
        _gwait(2, xc, sgc)
        _s(2, xc, ssc)
        _swait(1, xb, ssb)
        _g(4, xb, sgb)

        def _triple(j, _):
            for i in range(3):
                k = i
                ch = 3 * j + i
                _gwait(ch, bufs[k], gsems[k])
                _s(ch, bufs[k], ssems[k])
                km1 = (i + 2) % 3
                _swait(ch - 1, bufs[km1], ssems[km1])
                _g(ch + 2, bufs[km1], gsems[km1])
            return 0

        lax.fori_loop(1, 41, _triple, 0)
        _gwait(123, xa, sga)
        _s(123, xa, ssa)
        _swait(122, xc, ssc)
        _gwait(124, xb, sgb)
        _s(124, xb, ssb)
        _swait(123, xa, ssa)
        _swait(124, xb, ssb)
        plsc.subcore_barrier()

    with jax.named_scope("writeout"):
        def _wout(j, _):
            r0 = base_row + j * WROWS
            pltpu.async_copy(acc.at[pl.ds(r0, WROWS)],
                             out_hbm.at[c, pl.ds(r0, WROWS)], ssa)
            return 0

        nw = ROWS_PER_TILE // WROWS
        lax.fori_loop(0, nw, _wout, 0)

        def _wdrain(j, _):
            pltpu.make_async_copy(
                acc.at[pl.ds(base_row, WROWS)],
                out_hbm.at[c, pl.ds(base_row, WROWS)], ssa).wait()
            return 0

        lax.fori_loop(0, nw, _wdrain, 0)


def _combine_body(p_ref, o_ref):
    o_ref[...] = p_ref[0] + p_ref[1]


_combine_tc = pl.pallas_call(
    _combine_body,
    grid=(10,),
    in_specs=[pl.BlockSpec((2, N_NODES // 10, D_FEAT), lambda i: (0, i, 0))],
    out_specs=pl.BlockSpec((N_NODES // 10, D_FEAT), lambda i: (i, 0)),
    out_shape=jax.ShapeDtypeStruct((N_NODES, D_FEAT), jnp.float32),
)


def kernel(x, edge_index):
    ei = edge_index.astype(jnp.int32).reshape(2, NW, NCHUNK, CHUNK)
    partials = _gather_scatter_sc(x, ei)
    return _combine_tc(partials)
